# initial kernel scaffold (unmeasured)
import numpy as np
import jax
import jax.numpy as jnp
from jax import lax
from jax.experimental import pallas as pl
from jax.experimental.pallas import tpu as pltpu

ND = 32
B, S, D = 2, 128, 512
DH = 64
R = B * S
CH = R // ND


def kernel(x, Wq, Wk, Wv, Wo):
    d_local = Wq.shape[1]
    HL = d_local // DH

    inv = 1.0 / (10000.0 ** (np.arange(0, DH, 2) / DH))
    ang = np.arange(S)[:, None] * inv[None, :]
    cos = np.concatenate([np.cos(ang), np.cos(ang)], 1).astype(np.float32)
    sin = np.concatenate([np.sin(ang), np.sin(ang)], 1).astype(np.float32)

    perm = np.concatenate(
        [h * DH + np.concatenate([np.arange(0, DH, 2), np.arange(1, DH, 2)])
         for h in range(HL)]
    )
    Wq_p = Wq[:, perm]
    Wk_p = Wk[:, perm]

    def body(x_ref, wq_ref, wk_ref, wv_ref, wo_ref, cos_ref, sin_ref,
             out_ref, p_ref, acc_ref, comm_ref, send1, recv1, send2, recv2):
        me = lax.axis_index("i")

        bar = pltpu.get_barrier_semaphore()
        for d in range(1, ND):
            pl.semaphore_signal(
                bar, inc=1, device_id=((me + d) % ND,),
                device_id_type=pl.DeviceIdType.MESH,
            )
        pl.semaphore_wait(bar, ND - 1)

        cos_t = cos_ref[...]
        sin_t = sin_ref[...]

        def rope(t):
            half = jnp.concatenate([-t[:, DH // 2:], t[:, :DH // 2]], axis=1)
            return t * cos_t + half * sin_t

        for b in range(B):
            xb = x_ref[b]
            q = jnp.dot(xb, wq_ref[...], preferred_element_type=jnp.float32)
            k = jnp.dot(xb, wk_ref[...], preferred_element_type=jnp.float32)
            v = jnp.dot(xb, wv_ref[...], preferred_element_type=jnp.float32)
            ctxs = []
            for h in range(HL):
                qh = rope(q[:, h * DH:(h + 1) * DH])
                kh = rope(k[:, h * DH:(h + 1) * DH])
                s = jnp.dot(qh, kh.T, preferred_element_type=jnp.float32)
                s = s * 0.125
                s = s - jnp.max(s, axis=-1, keepdims=True)
                w = jnp.exp(s)
                w = w / jnp.sum(w, axis=-1, keepdims=True)
                ctxs.append(jnp.dot(w, v[:, h * DH:(h + 1) * DH],
                                    preferred_element_type=jnp.float32))
            ctx = jnp.concatenate(ctxs, axis=1)
            p_ref[pl.ds(b * S, S), :] = jnp.dot(
                ctx, wo_ref[...], preferred_element_type=jnp.float32)


        rdma1 = []
        for d in range(1, ND):
            j = (me + d) % ND
            snd = pltpu.make_async_remote_copy(
                src_ref=p_ref.at[pl.ds(j * CH, CH), :],
                dst_ref=comm_ref.at[d],
                send_sem=send1.at[d],
                recv_sem=recv1.at[d],
                device_id=(j,),
                device_id_type=pl.DeviceIdType.MESH,
            )
            snd.start()
            rdma1.append(snd)

        acc = p_ref[pl.ds(me * CH, CH), :]
        for d in range(1, ND):
            rdma1[d - 1].wait_recv()
            acc = acc + comm_ref[d]
        acc_ref[...] = acc
        out_ref[pl.ds(me * CH, CH), :] = acc

        rdma2 = []
        for d in range(1, ND):
            j = (me + d) % ND
            snd = pltpu.make_async_remote_copy(
                src_ref=acc_ref,
                dst_ref=out_ref.at[pl.ds(me * CH, CH), :],
                send_sem=send2.at[d],
                recv_sem=recv2.at[d],
                device_id=(j,),
                device_id_type=pl.DeviceIdType.MESH,
            )
            snd.start()
            rdma2.append(snd)

        for d in range(1, ND):
            rdma2[d - 1].wait_recv()
        for r in rdma1:
            r.wait_send()
        for r in rdma2:
            r.wait_send()

    out2 = pl.pallas_call(
        body,
        out_shape=jax.ShapeDtypeStruct((R, D), jnp.float32),
        in_specs=[pl.BlockSpec(memory_space=pltpu.VMEM)] * 7,
        out_specs=pl.BlockSpec(memory_space=pltpu.VMEM),
        scratch_shapes=[
            pltpu.VMEM((R, D), jnp.float32),
            pltpu.VMEM((CH, D), jnp.float32),
            pltpu.VMEM((ND, CH, D), jnp.float32),
            pltpu.SemaphoreType.DMA((ND,)),
            pltpu.SemaphoreType.DMA((ND,)),
            pltpu.SemaphoreType.DMA((ND,)),
            pltpu.SemaphoreType.DMA((ND,)),
        ],
        compiler_params=pltpu.CompilerParams(collective_id=0),
    )(x, Wq_p, Wk_p, Wv, jnp.asarray(cos), jnp.asarray(sin), Wo)
    return out2.reshape(B, S, D)


# baseline (device time: 35723 ns/iter reference)
import numpy as np
import jax
import jax.numpy as jnp
from jax import lax
from jax.experimental import pallas as pl
from jax.experimental.pallas import tpu as pltpu

ND = 32
B, S, D = 2, 128, 512
DH = 64
R = B * S
CH = R // ND


def kernel(x, Wq, Wk, Wv, Wo):
    d_local = Wq.shape[1]
    HL = d_local // DH

    inv = 1.0 / (10000.0 ** (np.arange(0, DH, 2) / DH))
    ang = np.arange(S)[:, None] * inv[None, :]
    cos = np.concatenate([np.cos(ang), np.cos(ang)], 1).astype(np.float32)
    sin = np.concatenate([np.sin(ang), np.sin(ang)], 1).astype(np.float32)

    perm = np.concatenate(
        [h * DH + np.concatenate([np.arange(0, DH, 2), np.arange(1, DH, 2)])
         for h in range(HL)]
    )
    Wq_p = Wq[:, perm]
    Wk_p = Wk[:, perm]

    def body(x_ref, wq_ref, wk_ref, wv_ref, wo_ref, cos_ref, sin_ref,
             out_ref, p_ref, acc_ref, comm_ref, send1, recv1, send2, recv2):
        me = lax.axis_index("i")

        bar = pltpu.get_barrier_semaphore()
        for d in range(1, ND):
            pl.semaphore_signal(
                bar, inc=1, device_id=((me + d) % ND,),
                device_id_type=pl.DeviceIdType.MESH,
            )
        pl.semaphore_wait(bar, ND - 1)

        cos_t = cos_ref[...]
        sin_t = sin_ref[...]

        def rope(t):
            half = jnp.concatenate([-t[:, DH // 2:], t[:, :DH // 2]], axis=1)
            return t * cos_t + half * sin_t

        for b in range(B):
            xb = x_ref[b]
            q = jnp.dot(xb, wq_ref[...], preferred_element_type=jnp.float32)
            k = jnp.dot(xb, wk_ref[...], preferred_element_type=jnp.float32)
            v = jnp.dot(xb, wv_ref[...], preferred_element_type=jnp.float32)
            ctxs = []
            for h in range(HL):
                qh = rope(q[:, h * DH:(h + 1) * DH])
                kh = rope(k[:, h * DH:(h + 1) * DH])
                s = jnp.dot(qh, kh.T, preferred_element_type=jnp.float32)
                s = s * 0.125
                s = s - jnp.max(s, axis=-1, keepdims=True)
                w = jnp.exp(s)
                w = w / jnp.sum(w, axis=-1, keepdims=True)
                ctxs.append(jnp.dot(w, v[:, h * DH:(h + 1) * DH],
                                    preferred_element_type=jnp.float32))
            ctx = jnp.concatenate(ctxs, axis=1)
            p_ref[pl.ds(b * S, S), :] = jnp.dot(
                ctx, wo_ref[...], preferred_element_type=jnp.float32)


        rdma1 = []
        for d in range(1, ND):
            j = (me + d) % ND
            snd = pltpu.make_async_remote_copy(
                src_ref=p_ref.at[pl.ds(j * CH, CH), :],
                dst_ref=comm_ref.at[d],
                send_sem=send1.at[d],
                recv_sem=recv1.at[d],
                device_id=(j,),
                device_id_type=pl.DeviceIdType.MESH,
            )
            snd.start()
            rdma1.append(snd)

        acc = p_ref[pl.ds(me * CH, CH), :]
        for d in range(1, ND):
            rdma1[d - 1].wait_recv()
            acc = acc + comm_ref[d]
        acc_ref[...] = acc
        out_ref[pl.ds(me * CH, CH), :] = acc

        rdma2 = []
        for d in range(1, ND):
            j = (me + d) % ND
            snd = pltpu.make_async_remote_copy(
                src_ref=acc_ref,
                dst_ref=out_ref.at[pl.ds(me * CH, CH), :],
                send_sem=send2.at[d],
                recv_sem=recv2.at[d],
                device_id=(j,),
                device_id_type=pl.DeviceIdType.MESH,
            )
            snd.start()
            rdma2.append(snd)

        for d in range(1, ND):
            rdma2[d - 1].wait_recv()
        for r in rdma1:
            r.wait_send()
        for r in rdma2:
            r.wait_send()

    out2 = pl.pallas_call(
        body,
        out_shape=jax.ShapeDtypeStruct((R, D), jnp.float32),
        in_specs=[pl.BlockSpec(memory_space=pltpu.VMEM)] * 7,
        out_specs=pl.BlockSpec(memory_space=pltpu.VMEM),
        scratch_shapes=[
            pltpu.VMEM((R, D), jnp.float32),
            pltpu.VMEM((CH, D), jnp.float32),
            pltpu.VMEM((ND, CH, D), jnp.float32),
            pltpu.SemaphoreType.DMA((ND,)),
            pltpu.SemaphoreType.DMA((ND,)),
            pltpu.SemaphoreType.DMA((ND,)),
            pltpu.SemaphoreType.DMA((ND,)),
        ],
        compiler_params=pltpu.CompilerParams(collective_id=0),
    )(x, Wq_p, Wk_p, Wv, Wo, jnp.asarray(cos), jnp.asarray(sin))
    return out2.reshape(B, S, D)


# device time: 35100 ns/iter; 1.0177x vs baseline; 1.0177x over previous
import numpy as np
import jax
import jax.numpy as jnp
from jax import lax
from jax.experimental import pallas as pl
from jax.experimental.pallas import tpu as pltpu

ND = 32
B, S, D = 2, 128, 512
DH = 64
R = B * S
CH = R // ND


def kernel(x, Wq, Wk, Wv, Wo):
    d_local = Wq.shape[1]
    HL = d_local // DH

    inv = 1.0 / (10000.0 ** (np.arange(0, DH, 2) / DH))
    ang = np.arange(S)[:, None] * inv[None, :]
    cos = np.concatenate([np.cos(ang), np.cos(ang)], 1).astype(np.float32)
    sin = np.concatenate([np.sin(ang), np.sin(ang)], 1).astype(np.float32)

    perm = np.concatenate(
        [h * DH + np.concatenate([np.arange(0, DH, 2), np.arange(1, DH, 2)])
         for h in range(HL)]
    )
    Wq_p = Wq[:, perm]
    Wk_p = Wk[:, perm]

    def body(x_ref, wq_ref, wk_ref, wv_ref, wo_ref, cos_ref, sin_ref,
             out_ref, p_ref, acc_ref, comm_ref, send1, recv1, send2, recv2):
        me = lax.axis_index("i")

        bar = pltpu.get_barrier_semaphore()
        for d in range(1, ND):
            pl.semaphore_signal(
                bar, inc=1, device_id=((me + d) % ND,),
                device_id_type=pl.DeviceIdType.MESH,
            )

        cos_t = cos_ref[...]
        sin_t = sin_ref[...]
        cosq_t = cos_t * 0.125
        sinq_t = sin_t * 0.125

        def rope(t, c, sn):
            half = jnp.concatenate([-t[:, DH // 2:], t[:, :DH // 2]], axis=1)
            return t * c + half * sn

        def attn_batch(b):
            xb = x_ref[b]
            q = jnp.dot(xb, wq_ref[...], preferred_element_type=jnp.float32)
            k = jnp.dot(xb, wk_ref[...], preferred_element_type=jnp.float32)
            v = jnp.dot(xb, wv_ref[...], preferred_element_type=jnp.float32)
            ctxs = []
            for h in range(HL):
                qh = rope(q[:, h * DH:(h + 1) * DH], cosq_t, sinq_t)
                kh = rope(k[:, h * DH:(h + 1) * DH], cos_t, sin_t)
                s = jnp.dot(qh, kh.T, preferred_element_type=jnp.float32)
                s = s - jnp.max(s, axis=-1, keepdims=True)
                w = jnp.exp(s)
                w = w / jnp.sum(w, axis=-1, keepdims=True)
                ctxs.append(jnp.dot(w, v[:, h * DH:(h + 1) * DH],
                                    preferred_element_type=jnp.float32))
            ctx = jnp.concatenate(ctxs, axis=1)
            p_ref[pl.ds(b * S, S), :] = jnp.dot(
                ctx, wo_ref[...], preferred_element_type=jnp.float32)


        rdma1 = []
        js = []
        for d in range(1, ND):
            j = (me + d) % ND
            js.append(j)
            rdma1.append(pltpu.make_async_remote_copy(
                src_ref=p_ref.at[pl.ds(j * CH, CH), :],
                dst_ref=comm_ref.at[d],
                send_sem=send1.at[d],
                recv_sem=recv1.at[d],
                device_id=(j,),
                device_id_type=pl.DeviceIdType.MESH,
            ))

        half_chunks = (B * S // 2) // CH

        attn_batch(0)
        pl.semaphore_wait(bar, ND - 1)
        for snd, j in zip(rdma1, js):
            @pl.when(j < half_chunks)
            def _():
                snd.start()

        attn_batch(1)
        for snd, j in zip(rdma1, js):
            @pl.when(j >= half_chunks)
            def _():
                snd.start()

        acc = p_ref[pl.ds(me * CH, CH), :]
        for d in range(1, ND):
            rdma1[d - 1].wait_recv()
            acc = acc + comm_ref[d]
        acc_ref[...] = acc
        out_ref[pl.ds(me * CH, CH), :] = acc

        rdma2 = []
        for d in range(1, ND):
            j = (me + d) % ND
            snd = pltpu.make_async_remote_copy(
                src_ref=acc_ref,
                dst_ref=out_ref.at[pl.ds(me * CH, CH), :],
                send_sem=send2.at[d],
                recv_sem=recv2.at[d],
                device_id=(j,),
                device_id_type=pl.DeviceIdType.MESH,
            )
            snd.start()
            rdma2.append(snd)

        for d in range(1, ND):
            rdma2[d - 1].wait_recv()
        for r in rdma1:
            r.wait_send()
        for r in rdma2:
            r.wait_send()

    out2 = pl.pallas_call(
        body,
        out_shape=jax.ShapeDtypeStruct((R, D), jnp.float32),
        in_specs=[pl.BlockSpec(memory_space=pltpu.VMEM)] * 7,
        out_specs=pl.BlockSpec(memory_space=pltpu.VMEM),
        scratch_shapes=[
            pltpu.VMEM((R, D), jnp.float32),
            pltpu.VMEM((CH, D), jnp.float32),
            pltpu.VMEM((ND, CH, D), jnp.float32),
            pltpu.SemaphoreType.DMA((ND,)),
            pltpu.SemaphoreType.DMA((ND,)),
            pltpu.SemaphoreType.DMA((ND,)),
            pltpu.SemaphoreType.DMA((ND,)),
        ],
        compiler_params=pltpu.CompilerParams(collective_id=0),
    )(x, Wq_p, Wk_p, Wv, Wo, jnp.asarray(cos), jnp.asarray(sin))
    return out2.reshape(B, S, D)
